# baseline (speedup 1.0000x reference)
import functools

import jax
import jax.numpy as jnp
from jax import lax
from jax.experimental import pallas as pl
from jax.experimental.pallas import tpu as pltpu
from jax.experimental.pallas import tpu_sc as plsc

VOCAB = 1000000
EMBED_DIM = 64
BATCH = 4096
SEQ = 200
SEQ_PAD = 208
HALF = SEQ_PAD // 2
NCHUNK = HALF // 16
DC = EMBED_DIM // 16
UNROLL = 8

_info = plsc.get_sparse_core_info()
NC = _info.num_cores
NS = _info.num_subcores
NW = NC * NS
DPW = BATCH // NW

_mesh = plsc.VectorSubcoreMesh(core_axis_name="c", subcore_axis_name="s")


def _count_half(idx_ref, d, h):
    cnt = jnp.zeros((16,), jnp.float32)
    for k in range(NCHUNK):
        v = idx_ref[d, h, pl.ds(k * 16, 16)]
        cnt = cnt + jnp.where(v != 0, 1.0, 0.0).astype(jnp.float32)
    tail = idx_ref[d, h, pl.ds(HALF - 16, 16)]
    lane = lax.iota(jnp.int32, 16)
    tcnt = jnp.where((lane >= 8) & (tail != 0), 1.0, 0.0).astype(jnp.float32)
    return jnp.sum(cnt + tcnt)


def _accum_half(rows_ref, acc):
    def body(i, acc):
        r = i * UNROLL
        for j in range(UNROLL):
            acc = tuple(
                acc[c] + rows_ref[r + j, pl.ds(c * 16, 16)] for c in range(DC)
            )
        return acc
    return lax.fori_loop(0, HALF // UNROLL, body, acc)


@functools.partial(
    pl.kernel,
    mesh=_mesh,
    out_type=jax.ShapeDtypeStruct((BATCH,), jnp.float32),
    scratch_types=[
        pltpu.VMEM((DPW, 2, HALF), jnp.int32),
        pltpu.VMEM((HALF, EMBED_DIM), jnp.float32),
        pltpu.VMEM((HALF, EMBED_DIM), jnp.float32),
        pltpu.VMEM((HALF, EMBED_DIM), jnp.float32),
        pltpu.VMEM((HALF, EMBED_DIM), jnp.float32),
        pltpu.VMEM((DPW,), jnp.float32),
        pltpu.VMEM((EMBED_DIM,), jnp.float32),
        pltpu.VMEM((EMBED_DIM,), jnp.float32),
        pltpu.VMEM((16,), jnp.float32),
        pltpu.SemaphoreType.DMA,
        pltpu.SemaphoreType.DMA,
    ],
    compiler_params=pltpu.CompilerParams(needs_layout_passes=False,
                                         use_tc_tiling_on_sc=False),
)
def _emb_pool_kernel(docs_hbm, table_hbm, w_hbm, b_hbm, out_hbm,
                     idx_v, rows00, rows01, rows10, rows11,
                     out_v, w_v, t0_v, b_v, sem0, sem1):
    wid = lax.axis_index("s") * NC + lax.axis_index("c")
    base = wid * DPW

    sets = ((rows00, rows01), (rows10, rows11))
    sems = (sem0, sem1)

    pltpu.sync_copy(docs_hbm.at[pl.ds(base, DPW)], idx_v)
    pltpu.sync_copy(w_hbm, w_v)
    pltpu.sync_copy(table_hbm.at[0], t0_v)
    pltpu.sync_copy(b_hbm, b_v)

    w = [w_v[pl.ds(c * 16, 16)] for c in range(DC)]
    t0 = [t0_v[pl.ds(c * 16, 16)] for c in range(DC)]
    bvec = b_v[pl.ds(0, 16)]
    lane = lax.iota(jnp.int32, 16)

    def issue(d, p):
        for h in range(2):
            pltpu.make_async_copy(
                table_hbm.at[idx_v.at[d, h]], sets[p][h], sems[p]).start()

    def drain(d, p):
        for h in range(2):
            pltpu.make_async_copy(
                table_hbm.at[idx_v.at[d, h]], sets[p][h], sems[p]).wait()

    issue(0, 0)
    issue(1, 1)

    def pair_body(i, carry):
        d0 = i * 2
        for p in range(2):
            d = d0 + p
            drain(d, p)

            acc = tuple(jnp.zeros((16,), jnp.float32) for _ in range(DC))
            acc = _accum_half(sets[p][0], acc)
            acc = _accum_half(sets[p][1], acc)

            @pl.when(d + 2 < DPW)
            def _():
                issue(d + 2, p)

            count = _count_half(idx_v, d, 0) + _count_half(idx_v, d, 1)
            countv = jnp.full((16,), count, jnp.float32)
            n0v = jnp.full((16,), jnp.float32(SEQ_PAD)) - countv
            invv = 1.0 / jnp.maximum(countv, jnp.full((16,), 1.0, jnp.float32))

            dot = jnp.zeros((16,), jnp.float32)
            for c in range(DC):
                pooled_c = (acc[c] - n0v * t0[c]) * invv
                dot = dot + pooled_c * w[c]
            lv = jnp.full((16,), jnp.sum(dot), jnp.float32) + bvec

            e = jnp.exp(lv - lv)
            val = e / e
            plsc.store_scatter(out_v, [jnp.full((16,), d, jnp.int32)], val,
                               mask=lane == 0)
        return carry

    lax.fori_loop(0, DPW // 2, pair_body, 0)
    pltpu.sync_copy(out_v, out_hbm.at[pl.ds(base, DPW)])


def kernel(documents, table, W, b):
    docs = jnp.pad(documents.astype(jnp.int32), ((0, 0), (0, SEQ_PAD - SEQ)))
    docs = docs.reshape(BATCH, 2, HALF)
    out = _emb_pool_kernel(docs, table, W.reshape(EMBED_DIM),
                           jnp.full((16,), b[0], jnp.float32))
    return out.reshape(BATCH, 1)


# speedup vs baseline: 1.5575x; 1.5575x over previous
import functools

import jax
import jax.numpy as jnp
from jax import lax
from jax.experimental import pallas as pl
from jax.experimental.pallas import tpu as pltpu
from jax.experimental.pallas import tpu_sc as plsc

VOCAB = 1000000
EMBED_DIM = 64
BATCH = 4096
SEQ = 200
S_A = 104
S_B = SEQ - S_A
NFULL = SEQ // 16
TAIL_OFF = SEQ - 16

_info = plsc.get_sparse_core_info()
NC = _info.num_cores
NS = _info.num_subcores
NW = NC * NS
DPW = BATCH // NW

G = 4
NG = DPW // G

_mesh = plsc.VectorSubcoreMesh(core_axis_name="c", subcore_axis_name="s")


def _matvec_body(x_ref, w_ref, o_ref):
    o_ref[...] = jnp.sum(x_ref[...] * w_ref[...], axis=1, keepdims=True)


_ROWS_PER_BLOCK = 8000


def _table_matvec(table, w_row):
    return pl.pallas_call(
        _matvec_body,
        grid=(VOCAB // _ROWS_PER_BLOCK,),
        in_specs=[
            pl.BlockSpec((_ROWS_PER_BLOCK, EMBED_DIM), lambda i: (i, 0)),
            pl.BlockSpec((1, EMBED_DIM), lambda i: (0, 0)),
        ],
        out_specs=pl.BlockSpec((_ROWS_PER_BLOCK, 1), lambda i: (i, 0)),
        out_shape=jax.ShapeDtypeStruct((VOCAB, 1), jnp.float32),
    )(table, w_row)


def _doc_reduce(ref, d, zero_is_pad):
    acc = jnp.zeros((16,), jnp.float32)
    for k in range(NFULL):
        v = ref[d, pl.ds(k * 16, 16)]
        if zero_is_pad:
            acc = acc + jnp.where(v != 0, 1.0, 0.0).astype(jnp.float32)
        else:
            acc = acc + v
    lane = lax.iota(jnp.int32, 16)
    tail = ref[d, pl.ds(TAIL_OFF, 16)]
    if zero_is_pad:
        t = jnp.where((lane >= 8) & (tail != 0), 1.0, 0.0).astype(jnp.float32)
    else:
        t = jnp.where(lane >= 8, tail, jnp.zeros((16,), jnp.float32))
    return jnp.sum(acc + t)


@functools.partial(
    pl.kernel,
    mesh=_mesh,
    out_type=jax.ShapeDtypeStruct((BATCH,), jnp.float32),
    scratch_types=[
        pltpu.VMEM((DPW, SEQ), jnp.int32),
        pltpu.VMEM((DPW, SEQ), jnp.float32),
        pltpu.VMEM((DPW,), jnp.float32),
        pltpu.VMEM((16,), jnp.float32),
        pltpu.VMEM((16,), jnp.float32),
        pltpu.VMEM_SHARED((VOCAB,), jnp.float32),
        pltpu.SemaphoreType.DMA,
        pltpu.SemaphoreType.DMA,
    ],
    compiler_params=pltpu.CompilerParams(needs_layout_passes=False,
                                         use_tc_tiling_on_sc=False),
)
def _pool_kernel(docs_hbm, tw_hbm, b_hbm, out_hbm,
                 idx_v, vals_v, out_v, tw0_v, b_v, tw_sh, sem0, sem1):
    cid = lax.axis_index("c")
    sid = lax.axis_index("s")
    wid = sid * NC + cid
    base = wid * DPW
    sems = (sem0, sem1)

    pltpu.sync_copy(docs_hbm.at[pl.ds(base, DPW)], idx_v)
    pltpu.sync_copy(tw_hbm.at[pl.ds(0, 16)], tw0_v)
    pltpu.sync_copy(b_hbm, b_v)

    @pl.when(sid == 0)
    def _():
        pltpu.sync_copy(tw_hbm, tw_sh)
    plsc.subcore_barrier()

    bvec = b_v[pl.ds(0, 16)]
    tw0 = jnp.full((16,), tw0_v[pl.ds(0, 16)][0], jnp.float32)
    lane = lax.iota(jnp.int32, 16)
    onev = jnp.full((16,), 1.0, jnp.float32)
    seqv = jnp.full((16,), jnp.float32(SEQ), jnp.float32)

    def _group_streams(g, p):
        for j in range(G):
            d = g * G + j
            for off, ln in ((0, S_A), (S_A, S_B)):
                yield pltpu.make_async_copy(
                    tw_sh.at[idx_v.at[d, pl.ds(off, ln)]],
                    vals_v.at[d, pl.ds(off, ln)],
                    sems[p])

    def issue(g, p):
        for cp in _group_streams(g, p):
            cp.start()

    def drain(g, p):
        for cp in _group_streams(g, p):
            cp.wait()

    def process(g):
        for j in range(G):
            d = g * G + j
            s = _doc_reduce(vals_v, d, zero_is_pad=False)
            count = _doc_reduce(idx_v, d, zero_is_pad=True)
            countv = jnp.full((16,), count, jnp.float32)
            n0v = seqv - countv
            invv = 1.0 / jnp.maximum(countv, onev)
            lv = (jnp.full((16,), s, jnp.float32) - n0v * tw0) * invv + bvec
            e = jnp.exp(lv - lv)
            val = e / e
            plsc.store_scatter(out_v, [jnp.full((16,), d, jnp.int32)], val,
                               mask=lane == 0)

    issue(0, 0)
    issue(1, 1)

    def pair_body(i, carry):
        g0 = i * 2
        for p in range(2):
            g = g0 + p
            drain(g, p)

            @pl.when(g + 2 < NG)
            def _():
                issue(g + 2, p)

            process(g)
        return carry

    lax.fori_loop(0, NG // 2, pair_body, 0)
    pltpu.sync_copy(out_v, out_hbm.at[pl.ds(base, DPW)])


def kernel(documents, table, W, b):
    tw = _table_matvec(table, W.reshape(1, EMBED_DIM)).reshape(VOCAB)
    out = _pool_kernel(documents.astype(jnp.int32), tw,
                       jnp.full((16,), b[0], jnp.float32))
    return out.reshape(BATCH, 1)


# speedup vs baseline: 7.7799x; 4.9951x over previous
import functools

import jax
import jax.numpy as jnp
from jax import lax
from jax.experimental import pallas as pl
from jax.experimental.pallas import tpu as pltpu
from jax.experimental.pallas import tpu_sc as plsc

VOCAB = 1000000
EMBED_DIM = 64
BATCH = 4096
SEQ = 200
S_A = 104
S_B = SEQ - S_A
NFULL = SEQ // 16
TAIL_OFF = SEQ - 16

_info = plsc.get_sparse_core_info()
NC = _info.num_cores
NS = _info.num_subcores
NW = NC * NS
DPW = BATCH // NW

G = 4
NG = DPW // G

_mesh = plsc.VectorSubcoreMesh(core_axis_name="c", subcore_axis_name="s")


def _matvec_body(xt_ref, w_ref, o_ref):
    o_ref[...] = jnp.sum(xt_ref[...] * w_ref[...], axis=0)


_ROWS_PER_BLOCK = 8192


def _table_matvec(table_t, w_col):
    return pl.pallas_call(
        _matvec_body,
        grid=(pl.cdiv(VOCAB, _ROWS_PER_BLOCK),),
        in_specs=[
            pl.BlockSpec((EMBED_DIM, _ROWS_PER_BLOCK), lambda i: (0, i)),
            pl.BlockSpec((EMBED_DIM, 1), lambda i: (0, 0)),
        ],
        out_specs=pl.BlockSpec((_ROWS_PER_BLOCK,), lambda i: (i,)),
        out_shape=jax.ShapeDtypeStruct((VOCAB,), jnp.float32),
    )(table_t, w_col)


def _doc_reduce(ref, d, zero_is_pad):
    acc = jnp.zeros((16,), jnp.float32)
    for k in range(NFULL):
        v = ref[d, pl.ds(k * 16, 16)]
        if zero_is_pad:
            acc = acc + jnp.where(v != 0, 1.0, 0.0).astype(jnp.float32)
        else:
            acc = acc + v
    lane = lax.iota(jnp.int32, 16)
    tail = ref[d, pl.ds(TAIL_OFF, 16)]
    if zero_is_pad:
        t = jnp.where((lane >= 8) & (tail != 0), 1.0, 0.0).astype(jnp.float32)
    else:
        t = jnp.where(lane >= 8, tail, jnp.zeros((16,), jnp.float32))
    return jnp.sum(acc + t)


@functools.partial(
    pl.kernel,
    mesh=_mesh,
    out_type=jax.ShapeDtypeStruct((BATCH,), jnp.float32),
    scratch_types=[
        pltpu.VMEM((DPW, SEQ), jnp.int32),
        pltpu.VMEM((DPW, SEQ), jnp.float32),
        pltpu.VMEM((DPW,), jnp.float32),
        pltpu.VMEM((16,), jnp.float32),
        pltpu.VMEM((16,), jnp.float32),
        pltpu.VMEM_SHARED((VOCAB,), jnp.float32),
        pltpu.SemaphoreType.DMA,
        pltpu.SemaphoreType.DMA,
    ],
    compiler_params=pltpu.CompilerParams(needs_layout_passes=False,
                                         use_tc_tiling_on_sc=False),
)
def _pool_kernel(docs_hbm, tw_hbm, b_hbm, out_hbm,
                 idx_v, vals_v, out_v, tw0_v, b_v, tw_sh, sem0, sem1):
    cid = lax.axis_index("c")
    sid = lax.axis_index("s")
    wid = sid * NC + cid
    base = wid * DPW
    sems = (sem0, sem1)

    pltpu.sync_copy(docs_hbm.at[pl.ds(base, DPW)], idx_v)
    pltpu.sync_copy(tw_hbm.at[pl.ds(0, 16)], tw0_v)
    pltpu.sync_copy(b_hbm, b_v)

    @pl.when(sid == 0)
    def _():
        pltpu.sync_copy(tw_hbm, tw_sh)
    plsc.subcore_barrier()

    bvec = b_v[pl.ds(0, 16)]
    tw0 = jnp.full((16,), tw0_v[pl.ds(0, 16)][0], jnp.float32)
    lane = lax.iota(jnp.int32, 16)
    onev = jnp.full((16,), 1.0, jnp.float32)
    seqv = jnp.full((16,), jnp.float32(SEQ), jnp.float32)

    def _group_streams(g, p):
        for j in range(G):
            d = g * G + j
            for off, ln in ((0, S_A), (S_A, S_B)):
                yield pltpu.make_async_copy(
                    tw_sh.at[idx_v.at[d, pl.ds(off, ln)]],
                    vals_v.at[d, pl.ds(off, ln)],
                    sems[p])

    def issue(g, p):
        for cp in _group_streams(g, p):
            cp.start()

    def drain(g, p):
        for cp in _group_streams(g, p):
            cp.wait()

    def process(g):
        for j in range(G):
            d = g * G + j
            s = _doc_reduce(vals_v, d, zero_is_pad=False)
            count = _doc_reduce(idx_v, d, zero_is_pad=True)
            countv = jnp.full((16,), count, jnp.float32)
            n0v = seqv - countv
            invv = 1.0 / jnp.maximum(countv, onev)
            lv = (jnp.full((16,), s, jnp.float32) - n0v * tw0) * invv + bvec
            e = jnp.exp(lv - lv)
            val = e / e
            plsc.store_scatter(out_v, [jnp.full((16,), d, jnp.int32)], val,
                               mask=lane == 0)

    issue(0, 0)
    issue(1, 1)

    def pair_body(i, carry):
        g0 = i * 2
        for p in range(2):
            g = g0 + p
            drain(g, p)

            @pl.when(g + 2 < NG)
            def _():
                issue(g + 2, p)

            process(g)
        return carry

    lax.fori_loop(0, NG // 2, pair_body, 0)
    pltpu.sync_copy(out_v, out_hbm.at[pl.ds(base, DPW)])


def kernel(documents, table, W, b):
    tw = _table_matvec(table.T, W)
    out = _pool_kernel(documents.astype(jnp.int32), tw,
                       jnp.full((16,), b[0], jnp.float32))
    return out.reshape(BATCH, 1)


# speedup vs baseline: 10.8264x; 1.3916x over previous
import functools

import jax
import jax.numpy as jnp
from jax import lax
from jax.experimental import pallas as pl
from jax.experimental.pallas import tpu as pltpu
from jax.experimental.pallas import tpu_sc as plsc

VOCAB = 1000000
EMBED_DIM = 64
BATCH = 4096
SEQ = 200
S_A = 104
S_B = SEQ - S_A
NFULL = SEQ // 16
TAIL_OFF = SEQ - 16

_info = plsc.get_sparse_core_info()
NC = _info.num_cores
NS = _info.num_subcores
NW = NC * NS
DPW = BATCH // NW

G = 4
NG = DPW // G

_mesh = plsc.VectorSubcoreMesh(core_axis_name="c", subcore_axis_name="s")


def _matvec_body(xt_ref, w_ref, o_ref):
    o_ref[...] = jnp.sum(xt_ref[...] * w_ref[...], axis=0)


_ROWS_PER_BLOCK = 32768


def _table_matvec(table_t, w_col):
    return pl.pallas_call(
        _matvec_body,
        grid=(pl.cdiv(VOCAB, _ROWS_PER_BLOCK),),
        in_specs=[
            pl.BlockSpec((EMBED_DIM, _ROWS_PER_BLOCK), lambda i: (0, i)),
            pl.BlockSpec((EMBED_DIM, 1), lambda i: (0, 0)),
        ],
        out_specs=pl.BlockSpec((_ROWS_PER_BLOCK,), lambda i: (i,)),
        out_shape=jax.ShapeDtypeStruct((VOCAB,), jnp.float32),
    )(table_t, w_col)


def _doc_reduce(ref, d, zero_is_pad):
    acc = jnp.zeros((16,), jnp.float32)
    for k in range(NFULL):
        v = ref[d, pl.ds(k * 16, 16)]
        if zero_is_pad:
            acc = acc + jnp.where(v != 0, 1.0, 0.0).astype(jnp.float32)
        else:
            acc = acc + v
    lane = lax.iota(jnp.int32, 16)
    tail = ref[d, pl.ds(TAIL_OFF, 16)]
    if zero_is_pad:
        t = jnp.where((lane >= 8) & (tail != 0), 1.0, 0.0).astype(jnp.float32)
    else:
        t = jnp.where(lane >= 8, tail, jnp.zeros((16,), jnp.float32))
    return jnp.sum(acc + t)


@functools.partial(
    pl.kernel,
    mesh=_mesh,
    out_type=jax.ShapeDtypeStruct((BATCH,), jnp.float32),
    scratch_types=[
        pltpu.VMEM((DPW, SEQ), jnp.int32),
        pltpu.VMEM((DPW, SEQ), jnp.float32),
        pltpu.VMEM((DPW,), jnp.float32),
        pltpu.VMEM((16,), jnp.float32),
        pltpu.VMEM((16,), jnp.float32),
        pltpu.VMEM_SHARED((VOCAB,), jnp.float32),
        pltpu.SemaphoreType.DMA,
        pltpu.SemaphoreType.DMA,
    ],
    compiler_params=pltpu.CompilerParams(needs_layout_passes=False,
                                         use_tc_tiling_on_sc=False),
)
def _pool_kernel(docs_hbm, tw_hbm, b_hbm, out_hbm,
                 idx_v, vals_v, out_v, tw0_v, b_v, tw_sh, sem0, sem1):
    cid = lax.axis_index("c")
    sid = lax.axis_index("s")
    wid = sid * NC + cid
    base = wid * DPW
    sems = (sem0, sem1)

    pltpu.sync_copy(docs_hbm.at[pl.ds(base, DPW)], idx_v)
    pltpu.sync_copy(tw_hbm.at[pl.ds(0, 16)], tw0_v)
    pltpu.sync_copy(b_hbm, b_v)

    @pl.when(sid == 0)
    def _():
        pltpu.sync_copy(tw_hbm, tw_sh)
    plsc.subcore_barrier()

    bvec = b_v[pl.ds(0, 16)]
    tw0 = jnp.full((16,), tw0_v[pl.ds(0, 16)][0], jnp.float32)
    lane = lax.iota(jnp.int32, 16)
    onev = jnp.full((16,), 1.0, jnp.float32)
    seqv = jnp.full((16,), jnp.float32(SEQ), jnp.float32)

    def _group_streams(g, p):
        for j in range(G):
            d = g * G + j
            for off, ln in ((0, S_A), (S_A, S_B)):
                yield pltpu.make_async_copy(
                    tw_sh.at[idx_v.at[d, pl.ds(off, ln)]],
                    vals_v.at[d, pl.ds(off, ln)],
                    sems[p])

    def issue(g, p):
        for cp in _group_streams(g, p):
            cp.start()

    def drain(g, p):
        for cp in _group_streams(g, p):
            cp.wait()

    def process(g):
        for j in range(G):
            d = g * G + j
            s = _doc_reduce(vals_v, d, zero_is_pad=False)
            count = _doc_reduce(idx_v, d, zero_is_pad=True)
            countv = jnp.full((16,), count, jnp.float32)
            n0v = seqv - countv
            invv = 1.0 / jnp.maximum(countv, onev)
            lv = (jnp.full((16,), s, jnp.float32) - n0v * tw0) * invv + bvec
            e = jnp.exp(lv - lv)
            val = e / e
            plsc.store_scatter(out_v, [jnp.full((16,), d, jnp.int32)], val,
                               mask=lane == 0)

    issue(0, 0)
    issue(1, 1)

    def pair_body(i, carry):
        g0 = i * 2
        for p in range(2):
            g = g0 + p
            drain(g, p)

            @pl.when(g + 2 < NG)
            def _():
                issue(g + 2, p)

            process(g)
        return carry

    lax.fori_loop(0, NG // 2, pair_body, 0)
    pltpu.sync_copy(out_v, out_hbm.at[pl.ds(base, DPW)])


def kernel(documents, table, W, b):
    tw = _table_matvec(table.T, W)
    out = _pool_kernel(documents.astype(jnp.int32), tw,
                       jnp.full((16,), b[0], jnp.float32))
    return out.reshape(BATCH, 1)
